# Initial kernel scaffold; baseline (speedup 1.0000x reference)
import jax
import jax.numpy as jnp
from jax.experimental import pallas as pl


def kernel(atomic_numbers, table):
    raise NotImplementedError("write your pallas kernel here")


# speedup vs baseline: 1.1960x; 1.1960x over previous
import functools

import jax
import jax.numpy as jnp
from jax import lax
from jax.experimental import pallas as pl
from jax.experimental.pallas import tpu as pltpu
from jax.experimental.pallas import tpu_sc as plsc

B = 50000
D = 256
CHUNK = 128
NW = 32
N_FULL = B // CHUNK
TAIL = B - N_FULL * CHUNK
BASE_CPW = N_FULL // NW
EXTRA = N_FULL - BASE_CPW * NW
MAX_CPW = BASE_CPW + 1
IDXBUF = MAX_CPW * CHUNK


def _gather_kernel(idx_hbm, table_hbm, out_hbm, idx_v, rows_v, sem):
    wid = lax.axis_index("s") * 2 + lax.axis_index("c")
    nc = BASE_CPW + jnp.where(wid < EXTRA, 1, 0)
    base_chunk = BASE_CPW * wid + jnp.minimum(wid, EXTRA)
    base_row = base_chunk * CHUNK

    pltpu.sync_copy(idx_hbm.at[pl.ds(base_row, BASE_CPW * CHUNK)],
                    idx_v.at[pl.ds(0, BASE_CPW * CHUNK)])

    @pl.when(wid < EXTRA)
    def _():
        pltpu.sync_copy(idx_hbm.at[pl.ds(base_row + BASE_CPW * CHUNK, CHUNK)],
                        idx_v.at[pl.ds(BASE_CPW * CHUNK, CHUNK)])

    @pl.when(wid == NW - 1)
    def _():
        pltpu.sync_copy(idx_hbm.at[pl.ds(N_FULL * CHUNK, TAIL)],
                        idx_v.at[pl.ds(BASE_CPW * CHUNK, TAIL)])

    def body(i, _):
        @pl.when(i < nc)
        def _():
            pltpu.async_copy(
                table_hbm.at[idx_v.at[pl.ds(i * CHUNK, CHUNK)]],
                rows_v, sem).wait()
            pltpu.sync_copy(rows_v,
                            out_hbm.at[pl.ds(base_row + i * CHUNK, CHUNK)])
        return 0

    lax.fori_loop(0, MAX_CPW, body, 0)

    @pl.when(wid == NW - 1)
    def _():
        pltpu.async_copy(
            table_hbm.at[idx_v.at[pl.ds(BASE_CPW * CHUNK, TAIL)]],
            rows_v.at[pl.ds(0, TAIL)], sem).wait()
        pltpu.sync_copy(rows_v.at[pl.ds(0, TAIL)],
                        out_hbm.at[pl.ds(N_FULL * CHUNK, TAIL)])


@jax.jit
def _run(atomic_numbers, table):
    mesh = plsc.VectorSubcoreMesh(core_axis_name="c", subcore_axis_name="s")
    f = functools.partial(
        pl.kernel, mesh=mesh,
        out_type=jax.ShapeDtypeStruct((B, D), jnp.float32),
        scratch_types=[
            pltpu.VMEM((IDXBUF,), jnp.int32),
            pltpu.VMEM((CHUNK, D), jnp.float32),
            pltpu.SemaphoreType.DMA,
        ],
    )(_gather_kernel)
    return f(atomic_numbers, table)


def kernel(atomic_numbers, table):
    return _run(atomic_numbers.astype(jnp.int32), table)
